# Initial kernel scaffold; baseline (speedup 1.0000x reference)
import jax
import jax.numpy as jnp
from jax.experimental import pallas as pl


def kernel(title_ids, title_tokens, title_table, text_table):
    raise NotImplementedError("write your pallas kernel here")


# speedup vs baseline: 7.7322x; 7.7322x over previous
import dataclasses
import functools

import jax
import jax.numpy as jnp
from jax import lax
from jax.experimental import pallas as pl
from jax.experimental.pallas import tpu as pltpu
from jax.experimental.pallas import tpu_sc as plsc

B = 16384
D = 64
L = 20
LANES = 16
NC, NS = 2, 16
NW = NC * NS
RPW = B // NW
CH = 32
NCHUNK = RPW // CH
TPC = CH * L

_CP = pltpu.CompilerParams()
if "needs_layout_passes" in pltpu.CompilerParams.__dataclass_fields__:
    _CP = dataclasses.replace(_CP, needs_layout_passes=False)
if "use_tc_tiling_on_sc" in pltpu.CompilerParams.__dataclass_fields__:
    _CP = dataclasses.replace(_CP, use_tc_tiling_on_sc=False)


@functools.partial(
    pl.kernel,
    compiler_params=_CP,
    out_type=jax.ShapeDtypeStruct((B, 2 * D), jnp.float32),
    mesh=plsc.VectorSubcoreMesh(core_axis_name="c", subcore_axis_name="s"),
    scratch_types=[
        pltpu.VMEM((CH,), jnp.int32),
        pltpu.VMEM((TPC,), jnp.int32),
        pltpu.VMEM((CH, D), jnp.float32),
        pltpu.VMEM((TPC, D), jnp.float32),
        pltpu.VMEM((CH, 2 * D), jnp.float32),
        pltpu.VMEM((1, D), jnp.float32),
        pltpu.VMEM((CH,), jnp.float32),
        pltpu.VMEM((CH,), jnp.float32),
        pltpu.SemaphoreType.DMA,
    ],
)
def _movie_sc(ids_hbm, toks_hbm, ttab_hbm, xtab_hbm, out_hbm,
              ids_v, toks_v, trows_v, krows_v, out_v, r0_v, inv_v, n0b_v,
              sem):
    wid = lax.axis_index("s") * NC + lax.axis_index("c")
    base = wid * RPW
    pltpu.sync_copy(xtab_hbm.at[pl.ds(0, 1), :], r0_v)
    r0 = [r0_v[0, pl.ds(k * LANES, LANES)] for k in range(D // LANES)]

    @pl.loop(0, NCHUNK)
    def _chunk(c):
        row0 = base + c * CH
        pltpu.sync_copy(ids_hbm.at[pl.ds(row0, CH)], ids_v)
        pltpu.sync_copy(toks_hbm.at[pl.ds(row0 * L, TPC)], toks_v)
        cps = [pltpu.async_copy(ttab_hbm.at[ids_v], trows_v, sem)]
        for i in range(TPC // 128):
            cps.append(pltpu.async_copy(
                xtab_hbm.at[toks_v.at[pl.ds(i * 128, 128)]],
                krows_v.at[pl.ds(i * 128, 128)], sem))

        for g in range(CH // LANES):
            cnt = jnp.zeros((LANES,), jnp.float32)
            rowv = lax.iota(jnp.int32, LANES) * L + (g * LANES * L)
            for t in range(L):
                tk = plsc.load_gather(toks_v, [rowv + t])
                cnt = cnt + jnp.where(tk != 0,
                                      jnp.float32(1.0), jnp.float32(0.0))
            inv = 1.0 / jnp.maximum(cnt, 1.0)
            inv_v[pl.ds(g * LANES, LANES)] = inv
            n0b_v[pl.ds(g * LANES, LANES)] = (jnp.float32(L) - cnt) * inv

        for cp in cps:
            cp.wait()

        @pl.loop(0, CH)
        def _row(r):
            bidx = jnp.full((LANES,), 0, jnp.int32) + r
            a = plsc.load_gather(inv_v, [bidx])
            nb = plsc.load_gather(n0b_v, [bidx])
            for k in range(D // LANES):
                sl = pl.ds(k * LANES, LANES)
                acc = krows_v[r * L, sl]
                for t in range(1, L):
                    acc = acc + krows_v[r * L + t, sl]
                out_v[r, sl] = trows_v[r, sl]
                out_v[r, pl.ds(D + k * LANES, LANES)] = acc * a - r0[k] * nb
        pltpu.sync_copy(out_v, out_hbm.at[pl.ds(row0, CH), :])


def kernel(title_ids, title_tokens, title_table, text_table):
    ids = jnp.asarray(title_ids, jnp.int32)
    toks = jnp.asarray(title_tokens, jnp.int32).reshape(-1)
    return _movie_sc(ids, toks,
                     jnp.asarray(title_table, jnp.float32),
                     jnp.asarray(text_table, jnp.float32))
